# Initial kernel scaffold; baseline (speedup 1.0000x reference)
import jax
import jax.numpy as jnp
from jax.experimental import pallas as pl


def kernel(word_embeds, edge_index, w_text, b_text, W_se1, b_se1, W_se2, b_se2, bn_g, bn_b, W_gate, b_gate, W_l1, b_l1, W_l2, b_l2, bn2_g, bn2_b, mask_w, mask_b):
    raise NotImplementedError("write your pallas kernel here")


# speedup vs baseline: 7.2492x; 7.2492x over previous
import functools

import jax
import jax.numpy as jnp
from jax import lax
from jax.experimental import pallas as pl
from jax.experimental.pallas import tpu as pltpu
from jax.experimental.pallas import tpu_sc as plsc

N = 10000
NP = 10240
H = 128
T = 5
E = 320000
VOCAB = 10000

NTILES = 16
NCORES = 2
CHUNK = 80
EPT = E // NTILES
NCHUNK = EPT // CHUNK
HCHUNK = NCHUNK // NCORES
NBLK = 5
BCHUNK = HCHUNK // NBLK
ROWS_PT = NP // NTILES

BM = 2000


def _mesh():
    return plsc.VectorSubcoreMesh(core_axis_name="c", subcore_axis_name="s")


@functools.partial(
    pl.kernel,
    out_type=jax.ShapeDtypeStruct((NCORES, NP, H), jnp.float32),
    mesh=_mesh(),
    scratch_types=[
        pltpu.VMEM((BCHUNK, CHUNK), jnp.int32),
        pltpu.VMEM((BCHUNK, CHUNK), jnp.int32),
        pltpu.VMEM((2, CHUNK, H), jnp.float32),
        pltpu.VMEM_SHARED((NP, H), jnp.float32),
        pltpu.SemaphoreType.DMA,
        pltpu.SemaphoreType.DMA,
    ],
)
def _sc_segsum(x_hbm, src_hbm, dst_hbm, zeros_hbm, out_hbm,
               src_v, dst_v, rows_v, acc, sem0, sem1):
    c = lax.axis_index("c")
    s = lax.axis_index("s")
    row0 = s * ROWS_PT
    pltpu.sync_copy(zeros_hbm.at[pl.ds(row0, ROWS_PT)], acc.at[pl.ds(row0, ROWS_PT)])
    first = True
    for blk in range(NBLK):
        pltpu.sync_copy(src_hbm.at[s].at[c].at[blk], src_v)
        pltpu.sync_copy(dst_hbm.at[s].at[c].at[blk], dst_v)
        if first:
            plsc.subcore_barrier()
            first = False

        def body(j0, _):
            d0 = pltpu.async_copy(x_hbm.at[src_v.at[j0]], rows_v.at[0], sem0)
            d1 = pltpu.async_copy(x_hbm.at[src_v.at[j0 + 1]], rows_v.at[1], sem1)
            d0.wait()
            pltpu.sync_copy(rows_v.at[0], acc.at[dst_v.at[j0]], add=True)
            d1.wait()
            pltpu.sync_copy(rows_v.at[1], acc.at[dst_v.at[j0 + 1]], add=True)
            return ()

        lax.fori_loop(0, BCHUNK // 2, lambda i, carry: body(i * 2, carry), (),
                      unroll=False)
        pltpu.async_copy(x_hbm.at[src_v.at[BCHUNK - 1]], rows_v.at[0], sem0).wait()
        pltpu.sync_copy(rows_v.at[0], acc.at[dst_v.at[BCHUNK - 1]], add=True)
    plsc.subcore_barrier()
    pltpu.sync_copy(acc.at[pl.ds(row0, ROWS_PT)],
                    out_hbm.at[c].at[pl.ds(row0, ROWS_PT)])


@functools.partial(
    pl.kernel,
    out_type=jax.ShapeDtypeStruct((NCORES, T * NP), jnp.float32),
    mesh=_mesh(),
    scratch_types=[
        pltpu.VMEM((HCHUNK, CHUNK), jnp.int32),
        pltpu.VMEM((CHUNK,), jnp.float32),
        pltpu.VMEM_SHARED((T * NP,), jnp.float32),
    ],
)
def _sc_degrees(dst_all, zeros_hbm, out_hbm, idx_v, ones_v, acc):
    c = lax.axis_index("c")
    s = lax.axis_index("s")
    for k in range(CHUNK // 16):
        ones_v[pl.ds(k * 16, 16)] = jnp.ones((16,), jnp.float32)

    @pl.when(s < T)
    def _():
        pltpu.sync_copy(zeros_hbm, acc.at[pl.ds(s * NP, NP)])

    plsc.subcore_barrier()
    for t in range(T):
        pltpu.sync_copy(dst_all.at[t].at[s].at[c], idx_v)

        def body(j, _):
            pltpu.sync_copy(ones_v, acc.at[idx_v.at[j]], add=True)
            return ()

        lax.fori_loop(0, HCHUNK, body, (), unroll=False)
    plsc.subcore_barrier()

    @pl.when(s < T)
    def _():
        pltpu.sync_copy(acc.at[pl.ds(s * NP, NP)], out_hbm.at[c].at[pl.ds(s * NP, NP)])


def _mm(a, b):
    return jnp.dot(a, b, preferred_element_type=jnp.float32)


def _dense_we(word_embeds, w_text, b_text):
    def body(a_ref, w_ref, b_ref, o_ref):
        o_ref[...] = _mm(a_ref[...], w_ref[...]) + b_ref[...]

    return pl.pallas_call(
        body,
        grid=(N // BM,),
        in_specs=[
            pl.BlockSpec((BM, 768), lambda i: (i, 0)),
            pl.BlockSpec((768, H), lambda i: (0, 0)),
            pl.BlockSpec((1, H), lambda i: (0, 0)),
        ],
        out_specs=pl.BlockSpec((BM, H), lambda i: (i, 0)),
        out_shape=jax.ShapeDtypeStruct((N, H), jnp.float32),
    )(word_embeds, w_text, b_text.reshape(1, H))


def _mm_bias(y, W, b):
    def body(y_ref, w_ref, b_ref, o_ref):
        o_ref[...] = _mm(y_ref[...], w_ref[...]) + b_ref[...]

    return pl.pallas_call(
        body,
        grid=(N // BM,),
        in_specs=[
            pl.BlockSpec((BM, H), lambda i: (i, 0)),
            pl.BlockSpec((H, H), lambda i: (0, 0)),
            pl.BlockSpec((1, H), lambda i: (0, 0)),
        ],
        out_specs=pl.BlockSpec((BM, H), lambda i: (i, 0)),
        out_shape=jax.ShapeDtypeStruct((N, H), jnp.float32),
    )(y, W, b.reshape(1, H))


def _norm_mm(s, dg, W, b):
    def body(s_ref, d_ref, w_ref, b_ref, o_ref):
        y = jnp.maximum((s_ref[0] + s_ref[1]) * d_ref[...], 0.0)
        o_ref[...] = _mm(y, w_ref[...]) + b_ref[...]

    return pl.pallas_call(
        body,
        grid=(N // BM,),
        in_specs=[
            pl.BlockSpec((NCORES, BM, H), lambda i: (0, i, 0)),
            pl.BlockSpec((BM, 1), lambda i: (i, 0)),
            pl.BlockSpec((H, H), lambda i: (0, 0)),
            pl.BlockSpec((1, H), lambda i: (0, 0)),
        ],
        out_specs=pl.BlockSpec((BM, H), lambda i: (i, 0)),
        out_shape=jax.ShapeDtypeStruct((N, H), jnp.float32),
    )(s, dg, W, b.reshape(1, H))


def _colstats(s, dg):
    def body(s_ref, d_ref, o_ref):
        t = jnp.maximum((s_ref[0] + s_ref[1]) * d_ref[...], 0.0)

        @pl.when(pl.program_id(0) == 0)
        def _():
            o_ref[...] = jnp.zeros_like(o_ref)

        o_ref[0:1, :] += jnp.sum(t, axis=0, keepdims=True)
        o_ref[1:2, :] += jnp.sum(t * t, axis=0, keepdims=True)

    return pl.pallas_call(
        body,
        grid=(N // BM,),
        in_specs=[
            pl.BlockSpec((NCORES, BM, H), lambda i: (0, i, 0)),
            pl.BlockSpec((BM, 1), lambda i: (i, 0)),
        ],
        out_specs=pl.BlockSpec((2, H), lambda i: (0, 0)),
        out_shape=jax.ShapeDtypeStruct((2, H), jnp.float32),
    )(s, dg)


def _bn_gate_mm(s, dg, stats, bn_g, bn_b, Wg_x, Wg_we, b_gate, we, Wn, bn_):
    def body(s_ref, d_ref, st_ref, g_ref, bb_ref, wgx_ref, wgw_ref, bg_ref,
             we_ref, wn_ref, bnn_ref, o_ref):
        t = jnp.maximum((s_ref[0] + s_ref[1]) * d_ref[...], 0.0)
        mean = st_ref[0:1, :] * (1.0 / N)
        ex2 = st_ref[1:2, :] * (1.0 / N)
        var = ex2 - mean * mean
        xb = (t - mean) * lax.rsqrt(var + 1e-5) * g_ref[...] + bb_ref[...]
        xb = jnp.maximum(xb, 0.0)
        wev = we_ref[...]
        gate = jax.nn.sigmoid(_mm(xb, wgx_ref[...]) + _mm(wev, wgw_ref[...])
                              + bg_ref[...])
        lx = gate * xb + (1.0 - gate) * wev
        o_ref[...] = _mm(lx, wn_ref[...]) + bnn_ref[...]

    return pl.pallas_call(
        body,
        grid=(N // BM,),
        in_specs=[
            pl.BlockSpec((NCORES, BM, H), lambda i: (0, i, 0)),
            pl.BlockSpec((BM, 1), lambda i: (i, 0)),
            pl.BlockSpec((2, H), lambda i: (0, 0)),
            pl.BlockSpec((1, H), lambda i: (0, 0)),
            pl.BlockSpec((1, H), lambda i: (0, 0)),
            pl.BlockSpec((H, H), lambda i: (0, 0)),
            pl.BlockSpec((H, H), lambda i: (0, 0)),
            pl.BlockSpec((1, H), lambda i: (0, 0)),
            pl.BlockSpec((BM, H), lambda i: (i, 0)),
            pl.BlockSpec((H, H), lambda i: (0, 0)),
            pl.BlockSpec((1, H), lambda i: (0, 0)),
        ],
        out_specs=pl.BlockSpec((BM, H), lambda i: (i, 0)),
        out_shape=jax.ShapeDtypeStruct((N, H), jnp.float32),
    )(s, dg, stats, bn_g.reshape(1, H), bn_b.reshape(1, H), Wg_x, Wg_we,
      b_gate.reshape(1, H), we, Wn, bn_.reshape(1, H))


def _last_mm1(s, dg, W_l2, b_l2):
    def body(s_ref, d_ref, w_ref, b_ref, o_ref):
        y = jnp.maximum((s_ref[0] + s_ref[1]) * d_ref[...], 0.0)
        z = _mm(y, w_ref[...]) + b_ref[...]
        o_ref[...] = jnp.broadcast_to(z, (BM, H))

    return pl.pallas_call(
        body,
        grid=(N // BM,),
        in_specs=[
            pl.BlockSpec((NCORES, BM, H), lambda i: (0, i, 0)),
            pl.BlockSpec((BM, 1), lambda i: (i, 0)),
            pl.BlockSpec((H, 1), lambda i: (0, 0)),
            pl.BlockSpec((1, 1), lambda i: (0, 0)),
        ],
        out_specs=pl.BlockSpec((BM, H), lambda i: (i, 0)),
        out_shape=jax.ShapeDtypeStruct((N, H), jnp.float32),
    )(s, dg, W_l2, b_l2.reshape(1, 1))


def _deginv(pdeg):
    def body(p_ref, o_ref):
        p = p_ref[...]
        o_ref[...] = 1.0 / jnp.maximum(p[0] + p[1], 1.0)

    return pl.pallas_call(
        body,
        grid=(1,),
        in_specs=[pl.BlockSpec((NCORES, T, N), lambda i: (0, 0, 0))],
        out_specs=pl.BlockSpec((T, N), lambda i: (0, 0)),
        out_shape=jax.ShapeDtypeStruct((T, N), jnp.float32),
    )(pdeg)


def _final_stats(p, dg):
    def body(p_ref, d_ref, o_ref):
        z = jnp.maximum((p_ref[0] + p_ref[1]) * d_ref[...], 0.0)

        @pl.when(pl.program_id(0) == 0)
        def _():
            o_ref[...] = jnp.zeros_like(o_ref)

        o_ref[0:1, :] += jnp.full((1, H), jnp.sum(z), jnp.float32)
        o_ref[1:2, :] += jnp.full((1, H), jnp.sum(z * z), jnp.float32)

    return pl.pallas_call(
        body,
        grid=(N // BM,),
        in_specs=[
            pl.BlockSpec((NCORES, BM, 1), lambda i: (0, i, 0)),
            pl.BlockSpec((BM, 1), lambda i: (i, 0)),
        ],
        out_specs=pl.BlockSpec((2, H), lambda i: (0, 0)),
        out_shape=jax.ShapeDtypeStruct((2, H), jnp.float32),
    )(p, dg)


def _final_apply(p, dg, stats, bn2_g, bn2_b, mask_w, mask_b):
    def body(p_ref, d_ref, st_ref, g_ref, b_ref, mw_ref, mb_ref, o_ref):
        z = jnp.maximum((p_ref[0] + p_ref[1]) * d_ref[...], 0.0)
        mean = st_ref[0, 0] * (1.0 / N)
        var = st_ref[1, 0] * (1.0 / N) - mean * mean
        xb = (z - mean) * lax.rsqrt(var + 1e-5) * g_ref[0, 0] + b_ref[0, 0]
        xb = jnp.maximum(xb, 0.0)
        o_ref[...] = jax.nn.sigmoid(xb * mw_ref[...] + mb_ref[...])

    return pl.pallas_call(
        body,
        grid=(N // BM,),
        in_specs=[
            pl.BlockSpec((NCORES, BM, 1), lambda i: (0, i, 0)),
            pl.BlockSpec((BM, 1), lambda i: (i, 0)),
            pl.BlockSpec((2, H), lambda i: (0, 0)),
            pl.BlockSpec((1, 1), lambda i: (0, 0)),
            pl.BlockSpec((1, 1), lambda i: (0, 0)),
            pl.BlockSpec((BM, 1), lambda i: (i, 0)),
            pl.BlockSpec((BM, 1), lambda i: (i, 0)),
        ],
        out_specs=pl.BlockSpec((BM, 1), lambda i: (i, 0)),
        out_shape=jax.ShapeDtypeStruct((VOCAB, 1), jnp.float32),
    )(p, dg, stats, bn2_g.reshape(1, 1), bn2_b.reshape(1, 1),
      mask_w.reshape(VOCAB, 1), mask_b.reshape(VOCAB, 1))


def kernel(word_embeds, edge_index, w_text, b_text, W_se1, b_se1, W_se2, b_se2,
           bn_g, bn_b, W_gate, b_gate, W_l1, b_l1, W_l2, b_l2, bn2_g, bn2_b,
           mask_w, mask_b):
    er = edge_index.astype(jnp.int32).reshape(
        T, 2, NTILES, NCORES, NBLK, BCHUNK, CHUNK)
    srcs = er[:, 0]
    dsts = er[:, 1]
    zeros128 = jnp.zeros((NP, H), jnp.float32)
    zerosNP = jnp.zeros((NP,), jnp.float32)

    dsts_deg = dsts.reshape(T, NTILES, NCORES, HCHUNK, CHUNK) + (
        jnp.arange(T, dtype=jnp.int32) * NP).reshape(T, 1, 1, 1, 1)
    pdeg = _sc_degrees(dsts_deg, zerosNP).reshape(NCORES, T, NP)[:, :, :N]
    dinv = _deginv(pdeg)

    we = _dense_we(word_embeds, w_text, b_text)
    x = _mm_bias(we, W_se1, b_se1)

    Wg_x = W_gate[:H]
    Wg_we = W_gate[H:]

    for i in range(T - 1):
        dg = dinv[i].reshape(N, 1)
        s1 = _sc_segsum(x, srcs[i], dsts[i], zeros128)[:, :N]
        x2 = _norm_mm(s1, dg, W_se2, b_se2)
        s2 = _sc_segsum(x2, srcs[i], dsts[i], zeros128)[:, :N]
        st = _colstats(s2, dg)
        Wn, bn_ = (W_se1, b_se1) if i < T - 2 else (W_l1, b_l1)
        x = _bn_gate_mm(s2, dg, st, bn_g, bn_b, Wg_x, Wg_we, b_gate,
                        we, Wn, bn_)

    dg4 = dinv[T - 1].reshape(N, 1)
    s1 = _sc_segsum(x, srcs[T - 1], dsts[T - 1], zeros128)[:, :N]
    zb = _last_mm1(s1, dg4, W_l2, b_l2)
    p = _sc_segsum(zb, srcs[T - 1], dsts[T - 1], zeros128)[:, :N, :1]
    st2 = _final_stats(p, dg4)
    return _final_apply(p, dg4, st2, bn2_g, bn2_b, mask_w, mask_b)


# speedup vs baseline: 7.5221x; 1.0376x over previous
import functools

import jax
import jax.numpy as jnp
from jax import lax
from jax.experimental import pallas as pl
from jax.experimental.pallas import tpu as pltpu
from jax.experimental.pallas import tpu_sc as plsc

N = 10000
NP = 10240
H = 128
T = 5
E = 320000
VOCAB = 10000

NTILES = 16
NCORES = 2
CHUNK = 80
EPT = E // NTILES
NCHUNK = EPT // CHUNK
HCHUNK = NCHUNK // NCORES
NBLK = 5
BCHUNK = HCHUNK // NBLK
ROWS_PT = NP // NTILES

BM = 2000


def _mesh():
    return plsc.VectorSubcoreMesh(core_axis_name="c", subcore_axis_name="s")


@functools.partial(
    pl.kernel,
    out_type=jax.ShapeDtypeStruct((NCORES, NP, H), jnp.float32),
    mesh=_mesh(),
    scratch_types=[
        pltpu.VMEM((BCHUNK, CHUNK), jnp.int32),
        pltpu.VMEM((BCHUNK, CHUNK), jnp.int32),
        pltpu.VMEM((2, CHUNK, H), jnp.float32),
        pltpu.VMEM_SHARED((NP, H), jnp.float32),
        pltpu.SemaphoreType.DMA,
        pltpu.SemaphoreType.DMA,
    ],
)
def _sc_segsum(x_hbm, src_hbm, dst_hbm, zeros_hbm, out_hbm,
               src_v, dst_v, rows_v, acc, sem0, sem1):
    c = lax.axis_index("c")
    s = lax.axis_index("s")
    row0 = s * ROWS_PT
    pltpu.sync_copy(zeros_hbm.at[pl.ds(row0, ROWS_PT)], acc.at[pl.ds(row0, ROWS_PT)])
    first = True
    for blk in range(NBLK):
        pltpu.sync_copy(src_hbm.at[s].at[c].at[blk], src_v)
        pltpu.sync_copy(dst_hbm.at[s].at[c].at[blk], dst_v)
        if first:
            plsc.subcore_barrier()
            first = False

        def body(j0, _):
            d0 = pltpu.async_copy(x_hbm.at[src_v.at[j0]], rows_v.at[0], sem0)
            d1 = pltpu.async_copy(x_hbm.at[src_v.at[j0 + 1]], rows_v.at[1], sem1)
            d0.wait()
            pltpu.sync_copy(rows_v.at[0], acc.at[dst_v.at[j0]], add=True)
            d1.wait()
            pltpu.sync_copy(rows_v.at[1], acc.at[dst_v.at[j0 + 1]], add=True)
            return ()

        lax.fori_loop(0, BCHUNK // 2, lambda i, carry: body(i * 2, carry), (),
                      unroll=False)
        pltpu.async_copy(x_hbm.at[src_v.at[BCHUNK - 1]], rows_v.at[0], sem0).wait()
        pltpu.sync_copy(rows_v.at[0], acc.at[dst_v.at[BCHUNK - 1]], add=True)
    plsc.subcore_barrier()
    pltpu.sync_copy(acc.at[pl.ds(row0, ROWS_PT)],
                    out_hbm.at[c].at[pl.ds(row0, ROWS_PT)])


@functools.partial(
    pl.kernel,
    out_type=jax.ShapeDtypeStruct((NCORES, T * NP), jnp.float32),
    mesh=_mesh(),
    scratch_types=[
        pltpu.VMEM((HCHUNK, CHUNK), jnp.int32),
        pltpu.VMEM((CHUNK,), jnp.float32),
        pltpu.VMEM_SHARED((T * NP,), jnp.float32),
    ],
)
def _sc_degrees(dst_all, zeros_hbm, out_hbm, idx_v, ones_v, acc):
    c = lax.axis_index("c")
    s = lax.axis_index("s")
    for k in range(CHUNK // 16):
        ones_v[pl.ds(k * 16, 16)] = jnp.ones((16,), jnp.float32)

    @pl.when(s < T)
    def _():
        pltpu.sync_copy(zeros_hbm, acc.at[pl.ds(s * NP, NP)])

    plsc.subcore_barrier()
    for t in range(T):
        pltpu.sync_copy(dst_all.at[t].at[s].at[c], idx_v)

        def body(j, _):
            pltpu.sync_copy(ones_v, acc.at[idx_v.at[j]], add=True)
            return ()

        lax.fori_loop(0, HCHUNK, body, (), unroll=False)
    plsc.subcore_barrier()

    @pl.when(s < T)
    def _():
        pltpu.sync_copy(acc.at[pl.ds(s * NP, NP)], out_hbm.at[c].at[pl.ds(s * NP, NP)])


def _mm(a, b):
    return jnp.dot(a, b, preferred_element_type=jnp.float32)


def _dense_we(word_embeds, w_text, b_text):
    def body(a_ref, w_ref, b_ref, o_ref):
        o_ref[...] = _mm(a_ref[...], w_ref[...]) + b_ref[...]

    return pl.pallas_call(
        body,
        grid=(N // BM,),
        in_specs=[
            pl.BlockSpec((BM, 768), lambda i: (i, 0)),
            pl.BlockSpec((768, H), lambda i: (0, 0)),
            pl.BlockSpec((1, H), lambda i: (0, 0)),
        ],
        out_specs=pl.BlockSpec((BM, H), lambda i: (i, 0)),
        out_shape=jax.ShapeDtypeStruct((N, H), jnp.float32),
    )(word_embeds, w_text, b_text.reshape(1, H))


def _mm_bias(y, W, b):
    def body(y_ref, w_ref, b_ref, o_ref):
        o_ref[...] = _mm(y_ref[...], w_ref[...]) + b_ref[...]

    return pl.pallas_call(
        body,
        grid=(N // BM,),
        in_specs=[
            pl.BlockSpec((BM, H), lambda i: (i, 0)),
            pl.BlockSpec((H, H), lambda i: (0, 0)),
            pl.BlockSpec((1, H), lambda i: (0, 0)),
        ],
        out_specs=pl.BlockSpec((BM, H), lambda i: (i, 0)),
        out_shape=jax.ShapeDtypeStruct((N, H), jnp.float32),
    )(y, W, b.reshape(1, H))


def _norm_mm(s, dg, W, b):
    def body(s_ref, d_ref, w_ref, b_ref, o_ref):
        y = jnp.maximum((s_ref[0] + s_ref[1]) * d_ref[...], 0.0)
        o_ref[...] = _mm(y, w_ref[...]) + b_ref[...]

    return pl.pallas_call(
        body,
        grid=(N // BM,),
        in_specs=[
            pl.BlockSpec((NCORES, BM, H), lambda i: (0, i, 0)),
            pl.BlockSpec((BM, 1), lambda i: (i, 0)),
            pl.BlockSpec((H, H), lambda i: (0, 0)),
            pl.BlockSpec((1, H), lambda i: (0, 0)),
        ],
        out_specs=pl.BlockSpec((BM, H), lambda i: (i, 0)),
        out_shape=jax.ShapeDtypeStruct((N, H), jnp.float32),
    )(s, dg, W, b.reshape(1, H))


def _colstats(s, dg):
    def body(s_ref, d_ref, o_ref):
        t = jnp.maximum((s_ref[0] + s_ref[1]) * d_ref[...], 0.0)

        @pl.when(pl.program_id(0) == 0)
        def _():
            o_ref[...] = jnp.zeros_like(o_ref)

        o_ref[0:1, :] += jnp.sum(t, axis=0, keepdims=True)
        o_ref[1:2, :] += jnp.sum(t * t, axis=0, keepdims=True)

    return pl.pallas_call(
        body,
        grid=(N // BM,),
        in_specs=[
            pl.BlockSpec((NCORES, BM, H), lambda i: (0, i, 0)),
            pl.BlockSpec((BM, 1), lambda i: (i, 0)),
        ],
        out_specs=pl.BlockSpec((2, H), lambda i: (0, 0)),
        out_shape=jax.ShapeDtypeStruct((2, H), jnp.float32),
    )(s, dg)


def _bn_gate_mm(s, dg, stats, bn_g, bn_b, Wg_x, Wg_we, b_gate, we, Wn, bn_):
    def body(s_ref, d_ref, st_ref, g_ref, bb_ref, wgx_ref, wgw_ref, bg_ref,
             we_ref, wn_ref, bnn_ref, o_ref):
        t = jnp.maximum((s_ref[0] + s_ref[1]) * d_ref[...], 0.0)
        mean = st_ref[0:1, :] * (1.0 / N)
        ex2 = st_ref[1:2, :] * (1.0 / N)
        var = ex2 - mean * mean
        xb = (t - mean) * lax.rsqrt(var + 1e-5) * g_ref[...] + bb_ref[...]
        xb = jnp.maximum(xb, 0.0)
        wev = we_ref[...]
        gate = jax.nn.sigmoid(_mm(xb, wgx_ref[...]) + _mm(wev, wgw_ref[...])
                              + bg_ref[...])
        lx = gate * xb + (1.0 - gate) * wev
        o_ref[...] = _mm(lx, wn_ref[...]) + bnn_ref[...]

    return pl.pallas_call(
        body,
        grid=(N // BM,),
        in_specs=[
            pl.BlockSpec((NCORES, BM, H), lambda i: (0, i, 0)),
            pl.BlockSpec((BM, 1), lambda i: (i, 0)),
            pl.BlockSpec((2, H), lambda i: (0, 0)),
            pl.BlockSpec((1, H), lambda i: (0, 0)),
            pl.BlockSpec((1, H), lambda i: (0, 0)),
            pl.BlockSpec((H, H), lambda i: (0, 0)),
            pl.BlockSpec((H, H), lambda i: (0, 0)),
            pl.BlockSpec((1, H), lambda i: (0, 0)),
            pl.BlockSpec((BM, H), lambda i: (i, 0)),
            pl.BlockSpec((H, H), lambda i: (0, 0)),
            pl.BlockSpec((1, H), lambda i: (0, 0)),
        ],
        out_specs=pl.BlockSpec((BM, H), lambda i: (i, 0)),
        out_shape=jax.ShapeDtypeStruct((N, H), jnp.float32),
    )(s, dg, stats, bn_g.reshape(1, H), bn_b.reshape(1, H), Wg_x, Wg_we,
      b_gate.reshape(1, H), we, Wn, bn_.reshape(1, H))


def _last_mm1(s, dg, W_l2, b_l2):
    def body(s_ref, d_ref, w_ref, b_ref, o_ref):
        y = jnp.maximum((s_ref[0] + s_ref[1]) * d_ref[...], 0.0)
        z = _mm(y, w_ref[...]) + b_ref[...]
        o_ref[...] = jnp.broadcast_to(z, (BM, H))

    return pl.pallas_call(
        body,
        grid=(N // BM,),
        in_specs=[
            pl.BlockSpec((NCORES, BM, H), lambda i: (0, i, 0)),
            pl.BlockSpec((BM, 1), lambda i: (i, 0)),
            pl.BlockSpec((H, 1), lambda i: (0, 0)),
            pl.BlockSpec((1, 1), lambda i: (0, 0)),
        ],
        out_specs=pl.BlockSpec((BM, H), lambda i: (i, 0)),
        out_shape=jax.ShapeDtypeStruct((N, H), jnp.float32),
    )(s, dg, W_l2, b_l2.reshape(1, 1))


def _deginv(pdeg):
    def body(p_ref, o_ref):
        p = p_ref[...]
        o_ref[...] = 1.0 / jnp.maximum(p[0] + p[1], 1.0)

    return pl.pallas_call(
        body,
        grid=(1,),
        in_specs=[pl.BlockSpec((NCORES, T, N), lambda i: (0, 0, 0))],
        out_specs=pl.BlockSpec((T, N), lambda i: (0, 0)),
        out_shape=jax.ShapeDtypeStruct((T, N), jnp.float32),
    )(pdeg)


def _final_stats(p, dg):
    def body(p_ref, d_ref, o_ref):
        z = jnp.maximum((p_ref[0] + p_ref[1]) * d_ref[...], 0.0)

        @pl.when(pl.program_id(0) == 0)
        def _():
            o_ref[...] = jnp.zeros_like(o_ref)

        o_ref[0:1, :] += jnp.full((1, H), jnp.sum(z), jnp.float32)
        o_ref[1:2, :] += jnp.full((1, H), jnp.sum(z * z), jnp.float32)

    return pl.pallas_call(
        body,
        grid=(N // BM,),
        in_specs=[
            pl.BlockSpec((NCORES, BM, 1), lambda i: (0, i, 0)),
            pl.BlockSpec((BM, 1), lambda i: (i, 0)),
        ],
        out_specs=pl.BlockSpec((2, H), lambda i: (0, 0)),
        out_shape=jax.ShapeDtypeStruct((2, H), jnp.float32),
    )(p, dg)


def _final_apply(p, dg, stats, bn2_g, bn2_b, mask_w, mask_b):
    def body(p_ref, d_ref, st_ref, g_ref, b_ref, mw_ref, mb_ref, o_ref):
        z = jnp.maximum((p_ref[0] + p_ref[1]) * d_ref[...], 0.0)
        mean = st_ref[0, 0] * (1.0 / N)
        var = st_ref[1, 0] * (1.0 / N) - mean * mean
        xb = (z - mean) * lax.rsqrt(var + 1e-5) * g_ref[0, 0] + b_ref[0, 0]
        xb = jnp.maximum(xb, 0.0)
        o_ref[...] = jax.nn.sigmoid(xb * mw_ref[...] + mb_ref[...])

    return pl.pallas_call(
        body,
        grid=(N // BM,),
        in_specs=[
            pl.BlockSpec((NCORES, BM, 1), lambda i: (0, i, 0)),
            pl.BlockSpec((BM, 1), lambda i: (i, 0)),
            pl.BlockSpec((2, H), lambda i: (0, 0)),
            pl.BlockSpec((1, 1), lambda i: (0, 0)),
            pl.BlockSpec((1, 1), lambda i: (0, 0)),
            pl.BlockSpec((BM, 1), lambda i: (i, 0)),
            pl.BlockSpec((BM, 1), lambda i: (i, 0)),
        ],
        out_specs=pl.BlockSpec((BM, 1), lambda i: (i, 0)),
        out_shape=jax.ShapeDtypeStruct((VOCAB, 1), jnp.float32),
    )(p, dg, stats, bn2_g.reshape(1, 1), bn2_b.reshape(1, 1),
      mask_w.reshape(VOCAB, 1), mask_b.reshape(VOCAB, 1))


def kernel(word_embeds, edge_index, w_text, b_text, W_se1, b_se1, W_se2, b_se2,
           bn_g, bn_b, W_gate, b_gate, W_l1, b_l1, W_l2, b_l2, bn2_g, bn2_b,
           mask_w, mask_b):
    er = edge_index.astype(jnp.int32).reshape(
        T, 2, NTILES, NCORES, NBLK, BCHUNK, CHUNK)
    srcs = er[:, 0]
    dsts = er[:, 1]
    zeros128 = jnp.zeros((NP, H), jnp.float32)
    zerosNP = jnp.zeros((NP,), jnp.float32)

    dsts_deg = dsts.reshape(T, NTILES, NCORES, HCHUNK, CHUNK) + (
        jnp.arange(T, dtype=jnp.int32) * NP).reshape(T, 1, 1, 1, 1)
    pdeg = _sc_degrees(dsts_deg, zerosNP).reshape(NCORES, T, NP)[:, :, :N]
    dinv = _deginv(pdeg)

    we = _dense_we(word_embeds, w_text, b_text)
    x = _mm_bias(we, W_se1, b_se1)

    Wg_x = W_gate[:H]
    Wg_we = W_gate[H:]

    for i in range(T - 1):
        dg = dinv[i].reshape(N, 1)
        s1 = _sc_segsum(x, srcs[i], dsts[i], zeros128)
        x2 = _norm_mm(s1, dg, W_se2, b_se2)
        s2 = _sc_segsum(x2, srcs[i], dsts[i], zeros128)
        st = _colstats(s2, dg)
        Wn, bn_ = (W_se1, b_se1) if i < T - 2 else (W_l1, b_l1)
        x = _bn_gate_mm(s2, dg, st, bn_g, bn_b, Wg_x, Wg_we, b_gate,
                        we, Wn, bn_)

    dg4 = dinv[T - 1].reshape(N, 1)
    s1 = _sc_segsum(x, srcs[T - 1], dsts[T - 1], zeros128)
    zb = _last_mm1(s1, dg4, W_l2, b_l2)
    p = _sc_segsum(zb, srcs[T - 1], dsts[T - 1], zeros128)[:, :N, :1]
    st2 = _final_stats(p, dg4)
    return _final_apply(p, dg4, st2, bn2_g, bn2_b, mask_w, mask_b)
